# initial kernel scaffold (unmeasured)
import jax
import jax.numpy as jnp
from jax import lax
from jax.experimental import pallas as pl
from jax.experimental.pallas import tpu as pltpu

N_DEV = 16


def kernel(x, w_mat):
    m, _ = x.shape
    _, n = w_mat.shape
    chunk = m // N_DEV

    def body(x_ref, w_ref, out_ref,
             acc_ref, send_ref, recv_ref, amax_ref,
             ring_send_sem, ring_recv_sems, amax_send_sems, amax_recv_sems):
        my = lax.axis_index("i")
        left = lax.rem(my + N_DEV - 1, N_DEV)
        right = lax.rem(my + 1, N_DEV)

        barrier_sem = pltpu.get_barrier_semaphore()
        for nbr in (left, right):
            pl.semaphore_signal(
                barrier_sem, inc=1,
                device_id=(nbr,), device_id_type=pl.DeviceIdType.MESH,
            )
        pl.semaphore_wait(barrier_sem, 2)

        acc_ref[...] = jnp.dot(
            x_ref[...].astype(jnp.bfloat16),
            w_ref[...].astype(jnp.bfloat16),
            preferred_element_type=jnp.float32,
        )

        def local_chunk(c):
            return acc_ref[pl.ds(c * chunk, chunk), :]

        for s in range(N_DEV - 1):
            sc = lax.rem(my - (s + 1) + 2 * N_DEV, N_DEV)
            if s == 0:
                send_f32 = local_chunk(sc)
            else:
                send_f32 = recv_ref[s - 1].astype(jnp.float32) + local_chunk(sc)
            send_ref[...] = send_f32.astype(jnp.bfloat16)
            rdma = pltpu.make_async_remote_copy(
                src_ref=send_ref,
                dst_ref=recv_ref.at[s],
                send_sem=ring_send_sem,
                recv_sem=ring_recv_sems.at[s],
                device_id=(right,),
                device_id_type=pl.DeviceIdType.MESH,
            )
            rdma.start()
            rdma.wait()

        y = recv_ref[N_DEV - 2].astype(jnp.float32) + local_chunk(my)
        out_ref[...] = y

        local_amax = jnp.max(jnp.abs(y))
        amax_ref[pl.ds(my, 1), :] = jnp.full((1, 128), local_amax,
                                             dtype=jnp.float32)
        descs = []
        for j in range(1, N_DEV):
            tgt = lax.rem(my + j, N_DEV)
            d = pltpu.make_async_remote_copy(
                src_ref=amax_ref.at[pl.ds(my, 1)],
                dst_ref=amax_ref.at[pl.ds(my, 1)],
                send_sem=amax_send_sems.at[j - 1],
                recv_sem=amax_recv_sems.at[j - 1],
                device_id=(tgt,),
                device_id_type=pl.DeviceIdType.MESH,
            )
            d.start()
            descs.append(d)
        for d in descs:
            d.wait_send()
        for d in descs:
            d.wait_recv()
        g_amax = jnp.max(amax_ref[...])

        scale = g_amax / 127.0
        q = jnp.clip(jnp.round(out_ref[...] / scale), -127.0, 127.0)
        out_ref[...] = q * scale

    return pl.pallas_call(
        body,
        out_shape=jax.ShapeDtypeStruct((chunk, n), jnp.float32),
        in_specs=[pl.BlockSpec(memory_space=pltpu.VMEM),
                  pl.BlockSpec(memory_space=pltpu.VMEM)],
        out_specs=pl.BlockSpec(memory_space=pltpu.VMEM),
        scratch_shapes=[
            pltpu.VMEM((m, n), jnp.float32),
            pltpu.VMEM((chunk, n), jnp.bfloat16),
            pltpu.VMEM((N_DEV - 1, chunk, n), jnp.bfloat16),
            pltpu.VMEM((N_DEV, 128), jnp.float32),
            pltpu.SemaphoreType.DMA,
            pltpu.SemaphoreType.DMA((N_DEV - 1,)),
            pltpu.SemaphoreType.DMA((N_DEV - 1,)),
            pltpu.SemaphoreType.DMA((N_DEV - 1,)),
        ],
        compiler_params=pltpu.CompilerParams(collective_id=0),
    )(x, w_mat)


# baseline (device time: 220672 ns/iter reference)
import jax
import jax.numpy as jnp
from jax import lax
from jax.experimental import pallas as pl
from jax.experimental.pallas import tpu as pltpu

N_DEV = 16


def kernel(x, w_mat):
    m, _ = x.shape
    _, n = w_mat.shape
    chunk = m // N_DEV

    def body(x_ref, w_ref, out_ref,
             acc_ref, send_ref, recv_ref, amax_ref,
             ring_send_sem, ring_recv_sems, amax_send_sems, amax_recv_sems):
        my = lax.axis_index("i")
        left = lax.rem(my + N_DEV - 1, N_DEV)
        right = lax.rem(my + 1, N_DEV)

        barrier_sem = pltpu.get_barrier_semaphore()
        for nbr in (left, right):
            pl.semaphore_signal(
                barrier_sem, inc=1,
                device_id=(nbr,), device_id_type=pl.DeviceIdType.MESH,
            )
        pl.semaphore_wait(barrier_sem, 2)

        acc_ref[...] = jnp.dot(
            x_ref[...].astype(jnp.bfloat16),
            w_ref[...].astype(jnp.bfloat16),
            preferred_element_type=jnp.float32,
        )

        def local_chunk(c):
            return acc_ref[pl.ds(c * chunk, chunk), :]

        for s in range(N_DEV - 1):
            sc = lax.rem(my - (s + 1) + 2 * N_DEV, N_DEV)
            if s == 0:
                send_f32 = local_chunk(sc)
            else:
                send_f32 = recv_ref[s - 1].astype(jnp.float32) + local_chunk(sc)
            send_ref[...] = send_f32.astype(jnp.bfloat16)
            rdma = pltpu.make_async_remote_copy(
                src_ref=send_ref,
                dst_ref=recv_ref.at[s],
                send_sem=ring_send_sem,
                recv_sem=ring_recv_sems.at[s],
                device_id=(right,),
                device_id_type=pl.DeviceIdType.MESH,
            )
            rdma.start()
            rdma.wait()

        y = recv_ref[N_DEV - 2].astype(jnp.float32) + local_chunk(my)
        out_ref[...] = y

        local_amax = jnp.max(jnp.abs(y))
        amax_ref[pl.ds(my, 1), :] = jnp.full((1, 128), local_amax,
                                             dtype=jnp.float32)
        descs = []
        for j in range(1, N_DEV):
            tgt = lax.rem(my + j, N_DEV)
            d = pltpu.make_async_remote_copy(
                src_ref=amax_ref.at[pl.ds(my, 1)],
                dst_ref=amax_ref.at[pl.ds(my, 1)],
                send_sem=amax_send_sems.at[j - 1],
                recv_sem=amax_recv_sems.at[j - 1],
                device_id=(tgt,),
                device_id_type=pl.DeviceIdType.MESH,
            )
            d.start()
            descs.append(d)
        for d in descs:
            d.wait_send()
        for d in descs:
            d.wait_recv()
        g_amax = jnp.max(amax_ref[...])

        scale = g_amax / 127.0
        q = jnp.clip(jnp.round(out_ref[...] / scale), -127.0, 127.0)
        out_ref[...] = q * scale

    return pl.pallas_call(
        body,
        out_shape=jax.ShapeDtypeStruct((chunk, n), jnp.float32),
        in_specs=[pl.BlockSpec(memory_space=pltpu.VMEM),
                  pl.BlockSpec(memory_space=pltpu.VMEM)],
        out_specs=pl.BlockSpec(memory_space=pltpu.VMEM),
        scratch_shapes=[
            pltpu.VMEM((m, n), jnp.float32),
            pltpu.VMEM((chunk, n), jnp.bfloat16),
            pltpu.VMEM((N_DEV - 1, chunk, n), jnp.bfloat16),
            pltpu.VMEM((N_DEV, 128), jnp.float32),
            pltpu.SemaphoreType.DMA,
            pltpu.SemaphoreType.DMA((N_DEV - 1,)),
            pltpu.SemaphoreType.DMA((N_DEV - 1,)),
            pltpu.SemaphoreType.DMA((N_DEV - 1,)),
        ],
        compiler_params=pltpu.CompilerParams(
            collective_id=0,
            vmem_limit_bytes=100 * 1024 * 1024,
        ),
    )(x, w_mat)


# device time: 137619 ns/iter; 1.6035x vs baseline; 1.6035x over previous
import jax
import jax.numpy as jnp
from jax import lax
from jax.experimental import pallas as pl
from jax.experimental.pallas import tpu as pltpu

N_DEV = 16
CW_HOPS = 8
CCW_HOPS = 7


def kernel(x, w_mat):
    m, _ = x.shape
    _, n = w_mat.shape
    chunk = m // N_DEV

    def body(x_ref, w_ref, out_ref,
             acc_ref, cw_ref, ccw_ref, amax_ref,
             cw_send_sem, cw_recv_sems, ccw_send_sem, ccw_recv_sems,
             amax_send_sems, amax_recv_sems):
        my = lax.axis_index("i")
        left = lax.rem(my + N_DEV - 1, N_DEV)
        right = lax.rem(my + 1, N_DEV)

        barrier_sem = pltpu.get_barrier_semaphore()
        for nbr in (left, right):
            pl.semaphore_signal(
                barrier_sem, inc=1,
                device_id=(nbr,), device_id_type=pl.DeviceIdType.MESH,
            )
        pl.semaphore_wait(barrier_sem, 2)

        acc_ref[...] = jnp.dot(
            x_ref[...].astype(jnp.bfloat16),
            w_ref[...].astype(jnp.bfloat16),
            preferred_element_type=jnp.float32,
        )

        def local_chunk(c):
            c = lax.rem(c + 2 * N_DEV, N_DEV)
            return acc_ref[pl.ds(c * chunk, chunk), :]

        cw_ref[0] = local_chunk(my + CW_HOPS).astype(jnp.bfloat16)
        ccw_ref[0] = local_chunk(my - CCW_HOPS).astype(jnp.bfloat16)

        for j in range(CW_HOPS):
            cw_rdma = pltpu.make_async_remote_copy(
                src_ref=cw_ref.at[j],
                dst_ref=cw_ref.at[j + 1],
                send_sem=cw_send_sem,
                recv_sem=cw_recv_sems.at[j],
                device_id=(right,),
                device_id_type=pl.DeviceIdType.MESH,
            )
            cw_rdma.start()
            if j < CCW_HOPS:
                ccw_rdma = pltpu.make_async_remote_copy(
                    src_ref=ccw_ref.at[j],
                    dst_ref=ccw_ref.at[j + 1],
                    send_sem=ccw_send_sem,
                    recv_sem=ccw_recv_sems.at[j],
                    device_id=(left,),
                    device_id_type=pl.DeviceIdType.MESH,
                )
                ccw_rdma.start()
            cw_rdma.wait()
            if j < CCW_HOPS:
                ccw_rdma.wait()
            if j < CW_HOPS - 1:
                cw_ref[j + 1] = (
                    cw_ref[j + 1].astype(jnp.float32)
                    + local_chunk(my + CW_HOPS - (j + 1))
                ).astype(jnp.bfloat16)
            if j < CCW_HOPS - 1:
                ccw_ref[j + 1] = (
                    ccw_ref[j + 1].astype(jnp.float32)
                    + local_chunk(my - CCW_HOPS + (j + 1))
                ).astype(jnp.bfloat16)

        y = (cw_ref[CW_HOPS].astype(jnp.float32)
             + ccw_ref[CCW_HOPS].astype(jnp.float32)
             + local_chunk(my))
        out_ref[...] = y

        local_amax = jnp.max(jnp.abs(y))
        amax_ref[pl.ds(my, 1), :] = jnp.full((1, 128), local_amax,
                                             dtype=jnp.float32)
        descs = []
        for j in range(1, N_DEV):
            tgt = lax.rem(my + j, N_DEV)
            d = pltpu.make_async_remote_copy(
                src_ref=amax_ref.at[pl.ds(my, 1)],
                dst_ref=amax_ref.at[pl.ds(my, 1)],
                send_sem=amax_send_sems.at[j - 1],
                recv_sem=amax_recv_sems.at[j - 1],
                device_id=(tgt,),
                device_id_type=pl.DeviceIdType.MESH,
            )
            d.start()
            descs.append(d)
        for d in descs:
            d.wait_send()
        for d in descs:
            d.wait_recv()
        g_amax = jnp.max(amax_ref[...])

        scale = g_amax / 127.0
        q = jnp.clip(jnp.round(out_ref[...] / scale), -127.0, 127.0)
        out_ref[...] = q * scale

    return pl.pallas_call(
        body,
        out_shape=jax.ShapeDtypeStruct((chunk, n), jnp.float32),
        in_specs=[pl.BlockSpec(memory_space=pltpu.VMEM),
                  pl.BlockSpec(memory_space=pltpu.VMEM)],
        out_specs=pl.BlockSpec(memory_space=pltpu.VMEM),
        scratch_shapes=[
            pltpu.VMEM((m, n), jnp.float32),
            pltpu.VMEM((CW_HOPS + 1, chunk, n), jnp.bfloat16),
            pltpu.VMEM((CCW_HOPS + 1, chunk, n), jnp.bfloat16),
            pltpu.VMEM((N_DEV, 128), jnp.float32),
            pltpu.SemaphoreType.DMA,
            pltpu.SemaphoreType.DMA((CW_HOPS,)),
            pltpu.SemaphoreType.DMA,
            pltpu.SemaphoreType.DMA((CCW_HOPS,)),
            pltpu.SemaphoreType.DMA((N_DEV - 1,)),
            pltpu.SemaphoreType.DMA((N_DEV - 1,)),
        ],
        compiler_params=pltpu.CompilerParams(
            collective_id=0,
            vmem_limit_bytes=100 * 1024 * 1024,
        ),
    )(x, w_mat)


# device time: 109463 ns/iter; 2.0160x vs baseline; 1.2572x over previous
import jax
import jax.numpy as jnp
from jax import lax
from jax.experimental import pallas as pl
from jax.experimental.pallas import tpu as pltpu

N_DEV = 16
CW_HOPS = 8
CCW_HOPS = 7


def kernel(x, w_mat):
    m, _ = x.shape
    _, n = w_mat.shape
    chunk = m // N_DEV
    half = n // 2

    def body(x_ref, w_ref, out_ref,
             acc_ref, cwa_ref, cwb_ref, ccwa_ref, ccwb_ref, amax_ref,
             cwa_ss, cwb_ss, ccwa_ss, ccwb_ss,
             cwa_rs, cwb_rs, ccwa_rs, ccwb_rs,
             amax_send_sems, amax_recv_sems):
        my = lax.axis_index("i")
        left = lax.rem(my + N_DEV - 1, N_DEV)
        right = lax.rem(my + 1, N_DEV)

        barrier_sem = pltpu.get_barrier_semaphore()
        for nbr in (left, right):
            pl.semaphore_signal(
                barrier_sem, inc=1,
                device_id=(nbr,), device_id_type=pl.DeviceIdType.MESH,
            )
        pl.semaphore_wait(barrier_sem, 2)

        def chunk_rows(c):
            c = lax.rem(c + 2 * N_DEV, N_DEV)
            return pl.ds(c * chunk, chunk)

        def local_chunk(c, c0):
            return acc_ref[chunk_rows(c), pl.ds(c0, half)]

        streams = [
            ("cwa", cwa_ref, cwa_ss, cwa_rs, CW_HOPS, right, 0),
            ("ccwa", ccwa_ref, ccwa_ss, ccwa_rs, CCW_HOPS, left, 0),
            ("cwb", cwb_ref, cwb_ss, cwb_rs, CW_HOPS, right, half),
            ("ccwb", ccwb_ref, ccwb_ss, ccwb_rs, CCW_HOPS, left, half),
        ]

        def chunk_of(name, j):
            if name.startswith("cw"):
                return my + CW_HOPS - j
            return my - CCW_HOPS + j

        def make_hop(buf, ss, rs, tgt, j):
            return pltpu.make_async_remote_copy(
                src_ref=buf.at[j],
                dst_ref=buf.at[j + 1],
                send_sem=ss,
                recv_sem=rs.at[j],
                device_id=(tgt,),
                device_id_type=pl.DeviceIdType.MESH,
            )

        for c in (my + CW_HOPS, my - CCW_HOPS):
            acc_ref[chunk_rows(c), :] = jnp.dot(
                x_ref[chunk_rows(c), :].astype(jnp.bfloat16),
                w_ref[...].astype(jnp.bfloat16),
                preferred_element_type=jnp.float32,
            )
        descs = {}
        for name, buf, ss, rs, hops, tgt, c0 in streams:
            buf[0] = local_chunk(chunk_of(name, 0), c0).astype(jnp.bfloat16)
            d = make_hop(buf, ss, rs, tgt, 0)
            d.start()
            descs[name] = [d]

        acc_ref[...] = jnp.dot(
            x_ref[...].astype(jnp.bfloat16),
            w_ref[...].astype(jnp.bfloat16),
            preferred_element_type=jnp.float32,
        )

        for j in range(1, CW_HOPS):
            for name, buf, ss, rs, hops, tgt, c0 in streams:
                if j >= hops:
                    continue
                prev = descs[name][j - 1]
                prev.wait_recv()
                buf[j] = (
                    buf[j].astype(jnp.float32)
                    + local_chunk(chunk_of(name, j), c0)
                ).astype(jnp.bfloat16)
                prev.wait_send()
                d = make_hop(buf, ss, rs, tgt, j)
                d.start()
                descs[name].append(d)

        for name, buf, ss, rs, hops, tgt, c0 in streams:
            descs[name][hops - 1].wait_recv()
        out_ref[:, pl.ds(0, half)] = (
            cwa_ref[CW_HOPS].astype(jnp.float32)
            + ccwa_ref[CCW_HOPS].astype(jnp.float32)
            + local_chunk(my, 0)
        )
        out_ref[:, pl.ds(half, half)] = (
            cwb_ref[CW_HOPS].astype(jnp.float32)
            + ccwb_ref[CCW_HOPS].astype(jnp.float32)
            + local_chunk(my, half)
        )
        for name, buf, ss, rs, hops, tgt, c0 in streams:
            descs[name][hops - 1].wait_send()

        local_amax = jnp.max(jnp.abs(out_ref[...]))
        amax_ref[pl.ds(my, 1), :] = jnp.full((1, 128), local_amax,
                                             dtype=jnp.float32)
        adescs = []
        for j in range(1, N_DEV):
            tgt = lax.rem(my + j, N_DEV)
            d = pltpu.make_async_remote_copy(
                src_ref=amax_ref.at[pl.ds(my, 1)],
                dst_ref=amax_ref.at[pl.ds(my, 1)],
                send_sem=amax_send_sems.at[j - 1],
                recv_sem=amax_recv_sems.at[j - 1],
                device_id=(tgt,),
                device_id_type=pl.DeviceIdType.MESH,
            )
            d.start()
            adescs.append(d)
        for d in adescs:
            d.wait_send()
        for d in adescs:
            d.wait_recv()
        g_amax = jnp.max(amax_ref[...])

        scale = g_amax / 127.0
        q = jnp.clip(jnp.round(out_ref[...] / scale), -127.0, 127.0)
        out_ref[...] = q * scale

    return pl.pallas_call(
        body,
        out_shape=jax.ShapeDtypeStruct((chunk, n), jnp.float32),
        in_specs=[pl.BlockSpec(memory_space=pltpu.VMEM),
                  pl.BlockSpec(memory_space=pltpu.VMEM)],
        out_specs=pl.BlockSpec(memory_space=pltpu.VMEM),
        scratch_shapes=[
            pltpu.VMEM((m, n), jnp.float32),
            pltpu.VMEM((CW_HOPS + 1, chunk, half), jnp.bfloat16),
            pltpu.VMEM((CW_HOPS + 1, chunk, half), jnp.bfloat16),
            pltpu.VMEM((CCW_HOPS + 1, chunk, half), jnp.bfloat16),
            pltpu.VMEM((CCW_HOPS + 1, chunk, half), jnp.bfloat16),
            pltpu.VMEM((N_DEV, 128), jnp.float32),
            pltpu.SemaphoreType.DMA,
            pltpu.SemaphoreType.DMA,
            pltpu.SemaphoreType.DMA,
            pltpu.SemaphoreType.DMA,
            pltpu.SemaphoreType.DMA((CW_HOPS,)),
            pltpu.SemaphoreType.DMA((CW_HOPS,)),
            pltpu.SemaphoreType.DMA((CCW_HOPS,)),
            pltpu.SemaphoreType.DMA((CCW_HOPS,)),
            pltpu.SemaphoreType.DMA((N_DEV - 1,)),
            pltpu.SemaphoreType.DMA((N_DEV - 1,)),
        ],
        compiler_params=pltpu.CompilerParams(
            collective_id=0,
            vmem_limit_bytes=100 * 1024 * 1024,
        ),
    )(x, w_mat)


# device time: 109462 ns/iter; 2.0160x vs baseline; 1.0000x over previous
import jax
import jax.numpy as jnp
from jax import lax
from jax.experimental import pallas as pl
from jax.experimental.pallas import tpu as pltpu

N_DEV = 16
CW_HOPS = 8
CCW_HOPS = 7


def kernel(x, w_mat):
    m, _ = x.shape
    _, n = w_mat.shape
    chunk = m // N_DEV
    half = n // 2

    def body(x_ref, w_ref, out_ref,
             acc_ref, cwa_ref, cwb_ref, ccwa_ref, ccwb_ref, amax_ref,
             cwa_ss, cwb_ss, ccwa_ss, ccwb_ss,
             cwa_rs, cwb_rs, ccwa_rs, ccwb_rs,
             amax_send_sems, amax_recv_sems):
        my = lax.axis_index("i")
        left = lax.rem(my + N_DEV - 1, N_DEV)
        right = lax.rem(my + 1, N_DEV)

        barrier_sem = pltpu.get_barrier_semaphore()
        for nbr in (left, right):
            pl.semaphore_signal(
                barrier_sem, inc=1,
                device_id=(nbr,), device_id_type=pl.DeviceIdType.MESH,
            )
        pl.semaphore_wait(barrier_sem, 2)

        def chunk_rows(c):
            c = lax.rem(c + 2 * N_DEV, N_DEV)
            return pl.ds(c * chunk, chunk)

        def local_chunk(c, c0):
            return acc_ref[chunk_rows(c), pl.ds(c0, half)].astype(jnp.float32)

        streams = [
            ("cwa", cwa_ref, cwa_ss, cwa_rs, CW_HOPS, right, 0),
            ("ccwa", ccwa_ref, ccwa_ss, ccwa_rs, CCW_HOPS, left, 0),
            ("cwb", cwb_ref, cwb_ss, cwb_rs, CW_HOPS, right, half),
            ("ccwb", ccwb_ref, ccwb_ss, ccwb_rs, CCW_HOPS, left, half),
        ]

        def chunk_of(name, j):
            if name.startswith("cw"):
                return my + CW_HOPS - j
            return my - CCW_HOPS + j

        def make_hop(buf, ss, rs, tgt, j):
            return pltpu.make_async_remote_copy(
                src_ref=buf.at[j],
                dst_ref=buf.at[j + 1],
                send_sem=ss,
                recv_sem=rs.at[j],
                device_id=(tgt,),
                device_id_type=pl.DeviceIdType.MESH,
            )

        for c in (my + CW_HOPS, my - CCW_HOPS):
            acc_ref[chunk_rows(c), :] = jnp.dot(
                x_ref[chunk_rows(c), :].astype(jnp.bfloat16),
                w_ref[...].astype(jnp.bfloat16),
                preferred_element_type=jnp.float32,
            ).astype(jnp.bfloat16)
        descs = {}
        for name, buf, ss, rs, hops, tgt, c0 in streams:
            d = pltpu.make_async_remote_copy(
                src_ref=acc_ref.at[chunk_rows(chunk_of(name, 0)),
                                   pl.ds(c0, half)],
                dst_ref=buf.at[1],
                send_sem=ss,
                recv_sem=rs.at[0],
                device_id=(tgt,),
                device_id_type=pl.DeviceIdType.MESH,
            )
            d.start()
            descs[name] = [d]

        acc_ref[...] = jnp.dot(
            x_ref[...].astype(jnp.bfloat16),
            w_ref[...].astype(jnp.bfloat16),
            preferred_element_type=jnp.float32,
        ).astype(jnp.bfloat16)

        for j in range(1, CW_HOPS):
            for name, buf, ss, rs, hops, tgt, c0 in streams:
                if j >= hops:
                    continue
                prev = descs[name][j - 1]
                prev.wait_recv()
                buf[j] = (
                    buf[j].astype(jnp.float32)
                    + local_chunk(chunk_of(name, j), c0)
                ).astype(jnp.bfloat16)
                prev.wait_send()
                d = make_hop(buf, ss, rs, tgt, j)
                d.start()
                descs[name].append(d)

        for name, buf, ss, rs, hops, tgt, c0 in streams:
            descs[name][hops - 1].wait_recv()
        out_ref[:, pl.ds(0, half)] = (
            cwa_ref[CW_HOPS].astype(jnp.float32)
            + ccwa_ref[CCW_HOPS].astype(jnp.float32)
            + local_chunk(my, 0)
        )
        out_ref[:, pl.ds(half, half)] = (
            cwb_ref[CW_HOPS].astype(jnp.float32)
            + ccwb_ref[CCW_HOPS].astype(jnp.float32)
            + local_chunk(my, half)
        )
        for name, buf, ss, rs, hops, tgt, c0 in streams:
            descs[name][hops - 1].wait_send()

        local_amax = jnp.max(jnp.abs(out_ref[...]))
        amax_ref[pl.ds(my, 1), :] = jnp.full((1, 128), local_amax,
                                             dtype=jnp.float32)
        adescs = []
        for j in range(1, N_DEV):
            tgt = lax.rem(my + j, N_DEV)
            d = pltpu.make_async_remote_copy(
                src_ref=amax_ref.at[pl.ds(my, 1)],
                dst_ref=amax_ref.at[pl.ds(my, 1)],
                send_sem=amax_send_sems.at[j - 1],
                recv_sem=amax_recv_sems.at[j - 1],
                device_id=(tgt,),
                device_id_type=pl.DeviceIdType.MESH,
            )
            d.start()
            adescs.append(d)
        for d in adescs:
            d.wait_send()
        for d in adescs:
            d.wait_recv()
        g_amax = jnp.max(amax_ref[...])

        scale = g_amax / 127.0
        q = jnp.clip(jnp.round(out_ref[...] / scale), -127.0, 127.0)
        out_ref[...] = q * scale

    return pl.pallas_call(
        body,
        out_shape=jax.ShapeDtypeStruct((chunk, n), jnp.float32),
        in_specs=[pl.BlockSpec(memory_space=pltpu.VMEM),
                  pl.BlockSpec(memory_space=pltpu.VMEM)],
        out_specs=pl.BlockSpec(memory_space=pltpu.VMEM),
        scratch_shapes=[
            pltpu.VMEM((m, n), jnp.bfloat16),
            pltpu.VMEM((CW_HOPS + 1, chunk, half), jnp.bfloat16),
            pltpu.VMEM((CW_HOPS + 1, chunk, half), jnp.bfloat16),
            pltpu.VMEM((CCW_HOPS + 1, chunk, half), jnp.bfloat16),
            pltpu.VMEM((CCW_HOPS + 1, chunk, half), jnp.bfloat16),
            pltpu.VMEM((N_DEV, 128), jnp.float32),
            pltpu.SemaphoreType.DMA,
            pltpu.SemaphoreType.DMA,
            pltpu.SemaphoreType.DMA,
            pltpu.SemaphoreType.DMA,
            pltpu.SemaphoreType.DMA((CW_HOPS,)),
            pltpu.SemaphoreType.DMA((CW_HOPS,)),
            pltpu.SemaphoreType.DMA((CCW_HOPS,)),
            pltpu.SemaphoreType.DMA((CCW_HOPS,)),
            pltpu.SemaphoreType.DMA((N_DEV - 1,)),
            pltpu.SemaphoreType.DMA((N_DEV - 1,)),
        ],
        compiler_params=pltpu.CompilerParams(
            collective_id=0,
            vmem_limit_bytes=100 * 1024 * 1024,
        ),
    )(x, w_mat)


# device time: 108876 ns/iter; 2.0268x vs baseline; 1.0054x over previous
import jax
import jax.numpy as jnp
from jax import lax
from jax.experimental import pallas as pl
from jax.experimental.pallas import tpu as pltpu

N_DEV = 16
CW_HOPS = 8
CCW_HOPS = 7


def kernel(x, w_mat):
    m, _ = x.shape
    _, n = w_mat.shape
    chunk = m // N_DEV
    half = n // 2

    def body(x_ref, w_ref, out_ref,
             xb_ref, wb_ref, cwa_ref, cwb_ref, ccwa_ref, ccwb_ref, amax_ref,
             cwa_ss, cwb_ss, ccwa_ss, ccwb_ss,
             cwa_rs, cwb_rs, ccwa_rs, ccwb_rs,
             amax_send_sems, amax_recv_sems):
        my = lax.axis_index("i")
        left = lax.rem(my + N_DEV - 1, N_DEV)
        right = lax.rem(my + 1, N_DEV)

        barrier_sem = pltpu.get_barrier_semaphore()
        for nbr in (left, right):
            pl.semaphore_signal(
                barrier_sem, inc=1,
                device_id=(nbr,), device_id_type=pl.DeviceIdType.MESH,
            )
        xb_ref[...] = x_ref[...].astype(jnp.bfloat16)
        wb_ref[...] = w_ref[...].astype(jnp.bfloat16)
        pl.semaphore_wait(barrier_sem, 2)

        def chunk_rows(c):
            c = lax.rem(c + 2 * N_DEV, N_DEV)
            return pl.ds(c * chunk, chunk)

        def ptile(c, c0):
            return jnp.dot(
                xb_ref[chunk_rows(c), :],
                wb_ref[:, pl.ds(c0, half)],
                preferred_element_type=jnp.float32,
            )

        streams = [
            ("cwa", cwa_ref, cwa_ss, cwa_rs, CW_HOPS, right, 0),
            ("ccwa", ccwa_ref, ccwa_ss, ccwa_rs, CCW_HOPS, left, 0),
            ("cwb", cwb_ref, cwb_ss, cwb_rs, CW_HOPS, right, half),
            ("ccwb", ccwb_ref, ccwb_ss, ccwb_rs, CCW_HOPS, left, half),
        ]

        def chunk_of(name, j):
            if name.startswith("cw"):
                return my + CW_HOPS - j
            return my - CCW_HOPS + j

        def make_hop(buf, ss, rs, tgt, j):
            return pltpu.make_async_remote_copy(
                src_ref=buf.at[j],
                dst_ref=buf.at[j + 1],
                send_sem=ss,
                recv_sem=rs.at[j],
                device_id=(tgt,),
                device_id_type=pl.DeviceIdType.MESH,
            )

        descs = {}
        for name, buf, ss, rs, hops, tgt, c0 in streams:
            buf[0] = ptile(chunk_of(name, 0), c0).astype(jnp.bfloat16)
            d = make_hop(buf, ss, rs, tgt, 0)
            d.start()
            descs[name] = [d]

        for j in range(1, CW_HOPS):
            for name, buf, ss, rs, hops, tgt, c0 in streams:
                if j >= hops:
                    continue
                prev = descs[name][j - 1]
                prev.wait_recv()
                buf[j] = (
                    buf[j].astype(jnp.float32) + ptile(chunk_of(name, j), c0)
                ).astype(jnp.bfloat16)
                prev.wait_send()
                d = make_hop(buf, ss, rs, tgt, j)
                d.start()
                descs[name].append(d)

        descs["cwa"][CW_HOPS - 1].wait_recv()
        descs["ccwa"][CCW_HOPS - 1].wait_recv()
        ya = (cwa_ref[CW_HOPS].astype(jnp.float32)
              + ccwa_ref[CCW_HOPS].astype(jnp.float32)
              + ptile(my, 0))
        out_ref[:, pl.ds(0, half)] = ya
        amax_a = jnp.max(jnp.abs(ya))
        descs["cwb"][CW_HOPS - 1].wait_recv()
        descs["ccwb"][CCW_HOPS - 1].wait_recv()
        yb = (cwb_ref[CW_HOPS].astype(jnp.float32)
              + ccwb_ref[CCW_HOPS].astype(jnp.float32)
              + ptile(my, half))
        out_ref[:, pl.ds(half, half)] = yb
        local_amax = jnp.maximum(amax_a, jnp.max(jnp.abs(yb)))

        amax_ref[pl.ds(my, 1), :] = jnp.full((1, 128), local_amax,
                                             dtype=jnp.float32)
        adescs = []
        for j in range(1, N_DEV):
            tgt = lax.rem(my + j, N_DEV)
            d = pltpu.make_async_remote_copy(
                src_ref=amax_ref.at[pl.ds(my, 1)],
                dst_ref=amax_ref.at[pl.ds(my, 1)],
                send_sem=amax_send_sems.at[j - 1],
                recv_sem=amax_recv_sems.at[j - 1],
                device_id=(tgt,),
                device_id_type=pl.DeviceIdType.MESH,
            )
            d.start()
            adescs.append(d)
        for name, buf, ss, rs, hops, tgt, c0 in streams:
            descs[name][hops - 1].wait_send()
        for d in adescs:
            d.wait_send()
        for d in adescs:
            d.wait_recv()
        g_amax = jnp.max(amax_ref[...])

        scale = g_amax / 127.0
        q = jnp.clip(jnp.round(out_ref[...] / scale), -127.0, 127.0)
        out_ref[...] = q * scale

    return pl.pallas_call(
        body,
        out_shape=jax.ShapeDtypeStruct((chunk, n), jnp.float32),
        in_specs=[pl.BlockSpec(memory_space=pltpu.VMEM),
                  pl.BlockSpec(memory_space=pltpu.VMEM)],
        out_specs=pl.BlockSpec(memory_space=pltpu.VMEM),
        scratch_shapes=[
            pltpu.VMEM((m, x.shape[1]), jnp.bfloat16),
            pltpu.VMEM((w_mat.shape[0], n), jnp.bfloat16),
            pltpu.VMEM((CW_HOPS + 1, chunk, half), jnp.bfloat16),
            pltpu.VMEM((CW_HOPS + 1, chunk, half), jnp.bfloat16),
            pltpu.VMEM((CCW_HOPS + 1, chunk, half), jnp.bfloat16),
            pltpu.VMEM((CCW_HOPS + 1, chunk, half), jnp.bfloat16),
            pltpu.VMEM((N_DEV, 128), jnp.float32),
            pltpu.SemaphoreType.DMA,
            pltpu.SemaphoreType.DMA,
            pltpu.SemaphoreType.DMA,
            pltpu.SemaphoreType.DMA,
            pltpu.SemaphoreType.DMA((CW_HOPS,)),
            pltpu.SemaphoreType.DMA((CW_HOPS,)),
            pltpu.SemaphoreType.DMA((CCW_HOPS,)),
            pltpu.SemaphoreType.DMA((CCW_HOPS,)),
            pltpu.SemaphoreType.DMA((N_DEV - 1,)),
            pltpu.SemaphoreType.DMA((N_DEV - 1,)),
        ],
        compiler_params=pltpu.CompilerParams(
            collective_id=0,
            vmem_limit_bytes=100 * 1024 * 1024,
        ),
    )(x, w_mat)


# device time: 108537 ns/iter; 2.0331x vs baseline; 1.0031x over previous
import jax
import jax.numpy as jnp
from jax import lax
from jax.experimental import pallas as pl
from jax.experimental.pallas import tpu as pltpu

N_DEV = 16
CW_HOPS = 8
CCW_HOPS = 7


def kernel(x, w_mat):
    m, _ = x.shape
    _, n = w_mat.shape
    chunk = m // N_DEV
    half = n // 2

    def body(x_ref, w_ref, out_ref,
             xb_ref, wb_ref, cwa_ref, cwb_ref, ccwa_ref, ccwb_ref, amax_ref,
             cwa_ss, cwb_ss, ccwa_ss, ccwb_ss,
             cwa_rs, cwb_rs, ccwa_rs, ccwb_rs,
             amax_send_sems, amax_recv_sems):
        my = lax.axis_index("i")
        left = lax.rem(my + N_DEV - 1, N_DEV)
        right = lax.rem(my + 1, N_DEV)

        barrier_sem = pltpu.get_barrier_semaphore()
        for nbr in (left, right):
            pl.semaphore_signal(
                barrier_sem, inc=1,
                device_id=(nbr,), device_id_type=pl.DeviceIdType.MESH,
            )
        xb_ref[...] = x_ref[...].astype(jnp.bfloat16)
        wb_ref[...] = w_ref[...].astype(jnp.bfloat16)

        def chunk_rows(c):
            c = lax.rem(c + 2 * N_DEV, N_DEV)
            return pl.ds(c * chunk, chunk)

        def ptile(c, c0):
            return jnp.dot(
                xb_ref[chunk_rows(c), :],
                wb_ref[:, pl.ds(c0, half)],
                preferred_element_type=jnp.float32,
            )

        streams = [
            ("cwa", cwa_ref, cwa_ss, cwa_rs, CW_HOPS, right, 0),
            ("ccwa", ccwa_ref, ccwa_ss, ccwa_rs, CCW_HOPS, left, 0),
            ("cwb", cwb_ref, cwb_ss, cwb_rs, CW_HOPS, right, half),
            ("ccwb", ccwb_ref, ccwb_ss, ccwb_rs, CCW_HOPS, left, half),
        ]

        def chunk_of(name, j):
            if name.startswith("cw"):
                return my + CW_HOPS - j
            return my - CCW_HOPS + j

        def make_hop(buf, ss, rs, tgt, j):
            return pltpu.make_async_remote_copy(
                src_ref=buf.at[j],
                dst_ref=buf.at[j + 1],
                send_sem=ss,
                recv_sem=rs.at[j],
                device_id=(tgt,),
                device_id_type=pl.DeviceIdType.MESH,
            )

        for name, buf, ss, rs, hops, tgt, c0 in streams:
            buf[0] = ptile(chunk_of(name, 0), c0).astype(jnp.bfloat16)
        pl.semaphore_wait(barrier_sem, 2)
        descs = {}
        for name, buf, ss, rs, hops, tgt, c0 in streams:
            d = make_hop(buf, ss, rs, tgt, 0)
            d.start()
            descs[name] = [d]

        for j in range(1, CW_HOPS):
            for name, buf, ss, rs, hops, tgt, c0 in streams:
                if j >= hops:
                    continue
                prev = descs[name][j - 1]
                prev.wait_recv()
                buf[j] = (
                    buf[j].astype(jnp.float32) + ptile(chunk_of(name, j), c0)
                ).astype(jnp.bfloat16)
                prev.wait_send()
                d = make_hop(buf, ss, rs, tgt, j)
                d.start()
                descs[name].append(d)

        descs["cwa"][CW_HOPS - 1].wait_recv()
        descs["ccwa"][CCW_HOPS - 1].wait_recv()
        ya = (cwa_ref[CW_HOPS].astype(jnp.float32)
              + ccwa_ref[CCW_HOPS].astype(jnp.float32)
              + ptile(my, 0))
        out_ref[:, pl.ds(0, half)] = ya
        amax_a = jnp.max(jnp.abs(ya))
        descs["cwb"][CW_HOPS - 1].wait_recv()
        descs["ccwb"][CCW_HOPS - 1].wait_recv()
        yb = (cwb_ref[CW_HOPS].astype(jnp.float32)
              + ccwb_ref[CCW_HOPS].astype(jnp.float32)
              + ptile(my, half))
        out_ref[:, pl.ds(half, half)] = yb
        local_amax = jnp.maximum(amax_a, jnp.max(jnp.abs(yb)))

        amax_ref[pl.ds(my, 1), :] = jnp.full((1, 128), local_amax,
                                             dtype=jnp.float32)
        adescs = []
        for j in range(1, N_DEV):
            tgt = lax.rem(my + j, N_DEV)
            d = pltpu.make_async_remote_copy(
                src_ref=amax_ref.at[pl.ds(my, 1)],
                dst_ref=amax_ref.at[pl.ds(my, 1)],
                send_sem=amax_send_sems.at[j - 1],
                recv_sem=amax_recv_sems.at[j - 1],
                device_id=(tgt,),
                device_id_type=pl.DeviceIdType.MESH,
            )
            d.start()
            adescs.append(d)
        for name, buf, ss, rs, hops, tgt, c0 in streams:
            descs[name][hops - 1].wait_send()
        for d in adescs:
            d.wait_send()
        for d in adescs:
            d.wait_recv()
        g_amax = jnp.max(amax_ref[...])

        scale = g_amax / 127.0
        q = jnp.clip(jnp.round(out_ref[...] / scale), -127.0, 127.0)
        out_ref[...] = q * scale

    return pl.pallas_call(
        body,
        out_shape=jax.ShapeDtypeStruct((chunk, n), jnp.float32),
        in_specs=[pl.BlockSpec(memory_space=pltpu.VMEM),
                  pl.BlockSpec(memory_space=pltpu.VMEM)],
        out_specs=pl.BlockSpec(memory_space=pltpu.VMEM),
        scratch_shapes=[
            pltpu.VMEM((m, x.shape[1]), jnp.bfloat16),
            pltpu.VMEM((w_mat.shape[0], n), jnp.bfloat16),
            pltpu.VMEM((CW_HOPS + 1, chunk, half), jnp.bfloat16),
            pltpu.VMEM((CW_HOPS + 1, chunk, half), jnp.bfloat16),
            pltpu.VMEM((CCW_HOPS + 1, chunk, half), jnp.bfloat16),
            pltpu.VMEM((CCW_HOPS + 1, chunk, half), jnp.bfloat16),
            pltpu.VMEM((N_DEV, 128), jnp.float32),
            pltpu.SemaphoreType.DMA,
            pltpu.SemaphoreType.DMA,
            pltpu.SemaphoreType.DMA,
            pltpu.SemaphoreType.DMA,
            pltpu.SemaphoreType.DMA((CW_HOPS,)),
            pltpu.SemaphoreType.DMA((CW_HOPS,)),
            pltpu.SemaphoreType.DMA((CCW_HOPS,)),
            pltpu.SemaphoreType.DMA((CCW_HOPS,)),
            pltpu.SemaphoreType.DMA((N_DEV - 1,)),
            pltpu.SemaphoreType.DMA((N_DEV - 1,)),
        ],
        compiler_params=pltpu.CompilerParams(
            collective_id=0,
            vmem_limit_bytes=100 * 1024 * 1024,
        ),
    )(x, w_mat)
